# baseline (speedup 1.0000x reference)
import functools

import jax
import jax.numpy as jnp
from jax import lax
from jax.experimental import pallas as pl
from jax.experimental.pallas import tpu as pltpu
from jax.experimental.pallas import tpu_sc as plsc

N_NODES = 10000
N_EDGES = 320000
IN_CH = 128
HEADS = 8
OUT_CH = 16
HID = HEADS * OUT_CH
EPS = 1e-06
LN_EPS = 1e-05

NC = 2
NS = 16
NW = NC * NS
EPT = N_EDGES // NW
CH = 40
NCHUNK = EPT // CH
N_PAD = 10240
ROWS_PT = N_PAD // NS
ZROWS = 64
TD = 2 * HID

_HIGH = jax.lax.Precision.HIGHEST

_GDN = lax.GatherDimensionNumbers(
    offset_dims=(), collapsed_slice_dims=(0,), start_index_map=(0,))


def _shuffle(v, idx):
    return lax.gather(v, idx[:, None], _GDN, (1,),
                      mode=lax.GatherScatterMode.PROMISE_IN_BOUNDS)


def _prep_body(x_ref, wlin_ref, s_ref, st_ref, hn_ref, h_ref):
    x = x_ref[...]
    h = lax.dot_general(x, wlin_ref[...], (((1,), (1,)), ((), ())),
                        precision=_HIGH, preferred_element_type=jnp.float32)
    h2s = lax.dot_general(h * h, s_ref[...], (((1,), (0,)), ((), ())),
                          precision=_HIGH, preferred_element_type=jnp.float32)
    nrm8 = jnp.maximum(jnp.sqrt(h2s), 1e-12)
    nrmx = lax.dot_general(nrm8, st_ref[...], (((1,), (0,)), ((), ())),
                           precision=_HIGH, preferred_element_type=jnp.float32)
    hn_ref[...] = h / nrmx
    h_ref[...] = h


def _prep(x, w_lin):
    R = 1000
    grid = N_NODES // R
    heads_of_ch = jnp.arange(IN_CH, dtype=jnp.int32) // OUT_CH
    sel = (heads_of_ch[:, None] == jnp.arange(HEADS)[None, :])
    S = sel.astype(jnp.float32)
    ST = S.T
    return pl.pallas_call(
        _prep_body,
        grid=(grid,),
        in_specs=[
            pl.BlockSpec((R, IN_CH), lambda i: (i, 0)),
            pl.BlockSpec((HID, IN_CH), lambda i: (0, 0)),
            pl.BlockSpec((IN_CH, HEADS), lambda i: (0, 0)),
            pl.BlockSpec((HEADS, IN_CH), lambda i: (0, 0)),
        ],
        out_specs=[
            pl.BlockSpec((R, HID), lambda i: (i, 0)),
            pl.BlockSpec((R, HID), lambda i: (i, 0)),
        ],
        out_shape=[
            jax.ShapeDtypeStruct((N_NODES, HID), jnp.float32),
            jax.ShapeDtypeStruct((N_NODES, HID), jnp.float32),
        ],
    )(x, w_lin, S, ST)


def _edge_body(tbl_hbm, hn_hbm, row_hbm, col_hbm, out_hbm,
               idxr, idxc, vs, vd, msg, acc, semi, semg, semd):
    cid = lax.axis_index("c")
    sid = lax.axis_index("s")
    wid = cid * NS + sid

    z16 = jnp.zeros((16,), jnp.float32)

    def zb(r, _):
        for hh in range(HID // 16):
            msg[0, r, pl.ds(hh * 16, 16)] = z16
        return 0

    lax.fori_loop(0, CH, zb, 0)
    for j in range(ROWS_PT // CH):
        pltpu.sync_copy(msg.at[0], acc.at[pl.ds(sid * ROWS_PT + j * CH, CH)])
    plsc.subcore_barrier()

    lanes = lax.iota(jnp.int32, 16)
    x8, x4, x2, x1 = (lanes ^ 8, lanes ^ 4, lanes ^ 2, lanes ^ 1)
    m8 = lanes < 8
    m4 = (lanes & 4) == 0
    m2 = (lanes & 2) == 0
    lane_of = (0, 8, 4, 12, 2, 10, 6, 14)
    bcast = [lax.iota(jnp.int32, 16) * 0 + lane_of[h] for h in range(HEADS)]

    def issue_idx(i, b):
        pltpu.async_copy(row_hbm.at[wid, i], idxr.at[b], semi.at[b])
        pltpu.async_copy(col_hbm.at[wid, i], idxc.at[b], semi.at[b])

    def wait_idx(i, b):
        pltpu.make_async_copy(row_hbm.at[wid, i], idxr.at[b],
                              semi.at[b]).wait()
        pltpu.make_async_copy(col_hbm.at[wid, i], idxc.at[b],
                              semi.at[b]).wait()

    def issue_gathers(b):
        pltpu.async_copy(tbl_hbm.at[idxc.at[b, 0]], vs.at[b], semg.at[b])
        pltpu.async_copy(hn_hbm.at[idxr.at[b, 0]], vd.at[b], semd.at[b])

    def wait_gathers(b):
        pltpu.make_async_copy(tbl_hbm.at[idxc.at[b, 0]], vs.at[b],
                              semg.at[b]).wait()
        pltpu.make_async_copy(hn_hbm.at[idxr.at[b, 0]], vd.at[b],
                              semd.at[b]).wait()

    def compute(b):
        @plsc.parallel_loop(0, CH, 1, unroll=4)
        def ebody(e):
            p = [vs[b, e, pl.ds(hh * 16, 16)] * vd[b, e, pl.ds(hh * 16, 16)]
                 for hh in range(HEADS)]
            r = [x + _shuffle(x, x8) for x in p]
            q = [jnp.where(m8, r[2 * k], r[2 * k + 1]) for k in range(4)]
            q = [x + _shuffle(x, x4) for x in q]
            u = [jnp.where(m4, q[0], q[1]), jnp.where(m4, q[2], q[3])]
            u = [x + _shuffle(x, x2) for x in u]
            v = jnp.where(m2, u[0], u[1])
            v = v + _shuffle(v, x1)
            v = jnp.minimum(jnp.maximum(v, EPS), 1.0)
            for hh in range(HEADS):
                hv = vs[b, e, pl.ds(HID + hh * 16, 16)]
                msg[b, e, pl.ds(hh * 16, 16)] = hv * _shuffle(v, bcast[hh])

    issue_idx(0, 0)
    wait_idx(0, 0)
    issue_gathers(0)

    def chunk_pair(j, _):
        for b in (0, 1):
            i = 2 * j + b
            b1 = 1 - b
            if b == 0:
                issue_idx(i + 1, b1)
                wait_gathers(b)
                wait_idx(i + 1, b1)
                issue_gathers(b1)
            else:
                @pl.when(j < NCHUNK // 2 - 1)
                def _():
                    issue_idx(i + 1, b1)
                    wait_idx(i + 1, b1)
                    issue_gathers(b1)
                wait_gathers(b)

            compute(b)
            pltpu.sync_copy(msg.at[b], acc.at[idxr.at[b, 0]], add=True)
        return 0

    lax.fori_loop(0, NCHUNK // 2, chunk_pair, 0)
    plsc.subcore_barrier()
    pltpu.sync_copy(acc.at[pl.ds(sid * ROWS_PT, ROWS_PT)],
                    out_hbm.at[cid, pl.ds(sid * ROWS_PT, ROWS_PT)])


def _edge(tbl, hn, row_idx, col_idx):
    mesh = plsc.VectorSubcoreMesh(core_axis_name="c", subcore_axis_name="s")
    f = functools.partial(
        pl.kernel,
        out_type=jax.ShapeDtypeStruct((NC, N_PAD, HID), jnp.float32),
        mesh=mesh,
        scratch_types=[
            pltpu.VMEM((2, 1, CH), jnp.int32),
            pltpu.VMEM((2, 1, CH), jnp.int32),
            pltpu.VMEM((2, CH, TD), jnp.float32),
            pltpu.VMEM((2, CH, HID), jnp.float32),
            pltpu.VMEM((2, CH, HID), jnp.float32),
            pltpu.VMEM_SHARED((N_PAD, HID), jnp.float32),
            pltpu.SemaphoreType.DMA((2,)),
            pltpu.SemaphoreType.DMA((2,)),
            pltpu.SemaphoreType.DMA((2,)),
        ],
    )(_edge_body)
    return f(tbl, hn, row_idx, col_idx)


def _fin_body(p_ref, w_ref, g_ref, b_ref, o_ref):
    oc = p_ref[0] + p_ref[1]
    w = w_ref[...]
    lin = lax.dot_general(oc, w, (((1,), (1,)), ((), ())),
                          precision=_HIGH, preferred_element_type=jnp.float32)
    ones = jnp.ones((1, HID), jnp.float32)
    wn2 = lax.dot_general(ones, w * w, (((1,), (1,)), ((), ())),
                          precision=_HIGH, preferred_element_type=jnp.float32)
    winv = 1.0 / jnp.maximum(jnp.sqrt(wn2), 1e-12)
    xn2 = jnp.sum(oc * oc, axis=1, keepdims=True)
    xinv = 1.0 / jnp.maximum(jnp.sqrt(xn2), 1e-12)
    cos2 = jnp.clip(lin * xinv * winv, EPS, 1.0)
    ob = lin * cos2
    mu = jnp.mean(ob, axis=1, keepdims=True)
    d = ob - mu
    var = jnp.mean(d * d, axis=1, keepdims=True)
    o_ref[...] = d * lax.rsqrt(var + LN_EPS) * g_ref[...] + b_ref[...]


def _finish(partials, w_bcos, g, b):
    R = 1000
    grid = N_NODES // R
    return pl.pallas_call(
        _fin_body,
        grid=(grid,),
        in_specs=[
            pl.BlockSpec((NC, R, HID), lambda i: (0, i, 0)),
            pl.BlockSpec((HID, HID), lambda i: (0, 0)),
            pl.BlockSpec((1, HID), lambda i: (0, 0)),
            pl.BlockSpec((1, HID), lambda i: (0, 0)),
        ],
        out_specs=pl.BlockSpec((R, HID), lambda i: (i, 0)),
        out_shape=jax.ShapeDtypeStruct((N_NODES, HID), jnp.float32),
    )(partials, w_bcos, g, b)


def kernel(x, edge_index, W_lin, W_bcos, ln_gamma, ln_beta):
    hn, h = _prep(x, W_lin)
    tbl = jnp.concatenate([hn, h], axis=1)
    row4 = edge_index[0].reshape(NW, NCHUNK, 1, CH)
    col4 = edge_index[1].reshape(NW, NCHUNK, 1, CH)
    partials = _edge(tbl, hn, row4, col4)
    return _finish(partials, W_bcos,
                   ln_gamma.reshape(1, HID), ln_beta.reshape(1, HID))


# speedup vs baseline: 2.1050x; 2.1050x over previous
import functools

import jax
import jax.numpy as jnp
from jax import lax
from jax.experimental import pallas as pl
from jax.experimental.pallas import tpu as pltpu
from jax.experimental.pallas import tpu_sc as plsc

N_NODES = 10000
N_EDGES = 320000
IN_CH = 128
HEADS = 8
OUT_CH = 16
HID = HEADS * OUT_CH
EPS = 1e-06
LN_EPS = 1e-05

NC = 2
NS = 16
NW = NC * NS
EPT = N_EDGES // NW
CH = 40
NCHUNK = EPT // CH
N_PAD = 10240
ROWS_PT = N_PAD // NS
ZROWS = 64
TD = 2 * HID

_HIGH = jax.lax.Precision.HIGHEST

_GDN = lax.GatherDimensionNumbers(
    offset_dims=(), collapsed_slice_dims=(0,), start_index_map=(0,))


def _shuffle(v, idx):
    return lax.gather(v, idx[:, None], _GDN, (1,),
                      mode=lax.GatherScatterMode.PROMISE_IN_BOUNDS)


def _prep_body(x_ref, wlin_ref, s_ref, st_ref, hn_ref, tbl_ref):
    x = x_ref[...]
    h = lax.dot_general(x, wlin_ref[...], (((1,), (1,)), ((), ())),
                        precision=_HIGH, preferred_element_type=jnp.float32)
    h2s = lax.dot_general(h * h, s_ref[...], (((1,), (0,)), ((), ())),
                          precision=_HIGH, preferred_element_type=jnp.float32)
    nrm8 = jnp.maximum(jnp.sqrt(h2s), 1e-12)
    nrmx = lax.dot_general(nrm8, st_ref[...], (((1,), (0,)), ((), ())),
                           precision=_HIGH, preferred_element_type=jnp.float32)
    hn = h / nrmx
    hn_ref[...] = hn
    tbl_ref[:, :HID] = hn
    tbl_ref[:, HID:] = h


def _prep(x, w_lin):
    R = 1000
    grid = N_NODES // R
    heads_of_ch = jnp.arange(IN_CH, dtype=jnp.int32) // OUT_CH
    sel = (heads_of_ch[:, None] == jnp.arange(HEADS)[None, :])
    S = sel.astype(jnp.float32)
    ST = S.T
    return pl.pallas_call(
        _prep_body,
        grid=(grid,),
        in_specs=[
            pl.BlockSpec((R, IN_CH), lambda i: (i, 0)),
            pl.BlockSpec((HID, IN_CH), lambda i: (0, 0)),
            pl.BlockSpec((IN_CH, HEADS), lambda i: (0, 0)),
            pl.BlockSpec((HEADS, IN_CH), lambda i: (0, 0)),
        ],
        out_specs=[
            pl.BlockSpec((R, HID), lambda i: (i, 0)),
            pl.BlockSpec((R, TD), lambda i: (i, 0)),
        ],
        out_shape=[
            jax.ShapeDtypeStruct((N_NODES, HID), jnp.float32),
            jax.ShapeDtypeStruct((N_NODES, TD), jnp.float32),
        ],
    )(x, w_lin, S, ST)


def _edge_body(tbl_hbm, hn_hbm, row_hbm, col_hbm, out_hbm,
               idxr, idxc, vs, vd, msg, acc, semi, semg, semd):
    cid = lax.axis_index("c")
    sid = lax.axis_index("s")
    wid = cid * NS + sid

    z16 = jnp.zeros((16,), jnp.float32)

    def zb(r, _):
        for hh in range(HID // 16):
            msg[0, r, pl.ds(hh * 16, 16)] = z16
        return 0

    lax.fori_loop(0, CH, zb, 0)
    for j in range(ROWS_PT // CH):
        pltpu.sync_copy(msg.at[0], acc.at[pl.ds(sid * ROWS_PT + j * CH, CH)])
    plsc.subcore_barrier()

    lanes = lax.iota(jnp.int32, 16)
    bfly = [lanes ^ (1 << k) for k in range(4)]

    def issue_idx(i, b):
        pltpu.async_copy(row_hbm.at[wid, i], idxr.at[b], semi.at[b])
        pltpu.async_copy(col_hbm.at[wid, i], idxc.at[b], semi.at[b])

    def wait_idx(i, b):
        pltpu.make_async_copy(row_hbm.at[wid, i], idxr.at[b],
                              semi.at[b]).wait()
        pltpu.make_async_copy(col_hbm.at[wid, i], idxc.at[b],
                              semi.at[b]).wait()

    def issue_gathers(b):
        pltpu.async_copy(tbl_hbm.at[idxc.at[b, 0]], vs.at[b], semg.at[b])
        pltpu.async_copy(hn_hbm.at[idxr.at[b, 0]], vd.at[b], semd.at[b])

    def wait_gathers(b):
        pltpu.make_async_copy(tbl_hbm.at[idxc.at[b, 0]], vs.at[b],
                              semg.at[b]).wait()
        pltpu.make_async_copy(hn_hbm.at[idxr.at[b, 0]], vd.at[b],
                              semd.at[b]).wait()

    def compute(b):
        @plsc.parallel_loop(0, CH, 1, unroll=4)
        def ebody(e):
            for hh in range(HEADS):
                sv = vs[b, e, pl.ds(hh * 16, 16)]
                dv = vd[b, e, pl.ds(hh * 16, 16)]
                hv = vs[b, e, pl.ds(HID + hh * 16, 16)]
                cos = sv * dv
                for idx in bfly:
                    cos = cos + _shuffle(cos, idx)
                scale = jnp.minimum(jnp.maximum(cos, EPS), 1.0)
                msg[b, e, pl.ds(hh * 16, 16)] = hv * scale

    issue_idx(0, 0)
    wait_idx(0, 0)
    issue_gathers(0)

    def chunk_pair(j, _):
        for b in (0, 1):
            i = 2 * j + b
            b1 = 1 - b
            if b == 0:
                issue_idx(i + 1, b1)
                wait_gathers(b)
                wait_idx(i + 1, b1)
                issue_gathers(b1)
            else:
                @pl.when(j < NCHUNK // 2 - 1)
                def _():
                    issue_idx(i + 1, b1)
                    wait_idx(i + 1, b1)
                    issue_gathers(b1)
                wait_gathers(b)

            compute(b)
            pltpu.sync_copy(msg.at[b], acc.at[idxr.at[b, 0]], add=True)
        return 0

    lax.fori_loop(0, NCHUNK // 2, chunk_pair, 0)
    plsc.subcore_barrier()
    pltpu.sync_copy(acc.at[pl.ds(sid * ROWS_PT, ROWS_PT)],
                    out_hbm.at[cid, pl.ds(sid * ROWS_PT, ROWS_PT)])


def _edge(tbl, hn, row_idx, col_idx):
    mesh = plsc.VectorSubcoreMesh(core_axis_name="c", subcore_axis_name="s")
    f = functools.partial(
        pl.kernel,
        out_type=jax.ShapeDtypeStruct((NC, N_PAD, HID), jnp.float32),
        mesh=mesh,
        scratch_types=[
            pltpu.VMEM((2, 1, CH), jnp.int32),
            pltpu.VMEM((2, 1, CH), jnp.int32),
            pltpu.VMEM((2, CH, TD), jnp.float32),
            pltpu.VMEM((2, CH, HID), jnp.float32),
            pltpu.VMEM((2, CH, HID), jnp.float32),
            pltpu.VMEM_SHARED((N_PAD, HID), jnp.float32),
            pltpu.SemaphoreType.DMA((2,)),
            pltpu.SemaphoreType.DMA((2,)),
            pltpu.SemaphoreType.DMA((2,)),
        ],
    )(_edge_body)
    return f(tbl, hn, row_idx, col_idx)


def _fin_body(p_ref, w_ref, g_ref, b_ref, o_ref):
    oc = p_ref[0] + p_ref[1]
    w = w_ref[...]
    lin = lax.dot_general(oc, w, (((1,), (1,)), ((), ())),
                          precision=_HIGH, preferred_element_type=jnp.float32)
    ones = jnp.ones((1, HID), jnp.float32)
    wn2 = lax.dot_general(ones, w * w, (((1,), (1,)), ((), ())),
                          precision=_HIGH, preferred_element_type=jnp.float32)
    winv = 1.0 / jnp.maximum(jnp.sqrt(wn2), 1e-12)
    xn2 = jnp.sum(oc * oc, axis=1, keepdims=True)
    xinv = 1.0 / jnp.maximum(jnp.sqrt(xn2), 1e-12)
    cos2 = jnp.clip(lin * xinv * winv, EPS, 1.0)
    ob = lin * cos2
    mu = jnp.mean(ob, axis=1, keepdims=True)
    d = ob - mu
    var = jnp.mean(d * d, axis=1, keepdims=True)
    o_ref[...] = d * lax.rsqrt(var + LN_EPS) * g_ref[...] + b_ref[...]


def _finish(partials, w_bcos, g, b):
    R = 1000
    grid = N_NODES // R
    return pl.pallas_call(
        _fin_body,
        grid=(grid,),
        in_specs=[
            pl.BlockSpec((NC, R, HID), lambda i: (0, i, 0)),
            pl.BlockSpec((HID, HID), lambda i: (0, 0)),
            pl.BlockSpec((1, HID), lambda i: (0, 0)),
            pl.BlockSpec((1, HID), lambda i: (0, 0)),
        ],
        out_specs=pl.BlockSpec((R, HID), lambda i: (i, 0)),
        out_shape=jax.ShapeDtypeStruct((N_NODES, HID), jnp.float32),
    )(partials, w_bcos, g, b)


def kernel(x, edge_index, W_lin, W_bcos, ln_gamma, ln_beta):
    hn, tbl = _prep(x, W_lin)
    row4 = edge_index[0].reshape(NW, NCHUNK, 1, CH)
    col4 = edge_index[1].reshape(NW, NCHUNK, 1, CH)
    partials = _edge(tbl, hn, row4, col4)
    return _finish(partials, W_bcos,
                   ln_gamma.reshape(1, HID), ln_beta.reshape(1, HID))


# speedup vs baseline: 2.3688x; 1.1253x over previous
import functools

import jax
import jax.numpy as jnp
from jax import lax
from jax.experimental import pallas as pl
from jax.experimental.pallas import tpu as pltpu
from jax.experimental.pallas import tpu_sc as plsc

N_NODES = 10000
N_EDGES = 320000
IN_CH = 128
HEADS = 8
OUT_CH = 16
HID = HEADS * OUT_CH
EPS = 1e-06
LN_EPS = 1e-05

NC = 2
NS = 16
NW = NC * NS
EPT = N_EDGES // NW
CH = 40
NCHUNK = EPT // CH
N_PAD = 10240
ROWS_PT = N_PAD // NS
ZROWS = 64
TD = 2 * HID

_HIGH = jax.lax.Precision.HIGHEST

_GDN = lax.GatherDimensionNumbers(
    offset_dims=(), collapsed_slice_dims=(0,), start_index_map=(0,))


def _shuffle(v, idx):
    return lax.gather(v, idx[:, None], _GDN, (1,),
                      mode=lax.GatherScatterMode.PROMISE_IN_BOUNDS)


def _prep_body(x_ref, wlin_ref, s_ref, st_ref, hn_ref, tbl_ref):
    x = x_ref[...]
    h = lax.dot_general(x, wlin_ref[...], (((1,), (1,)), ((), ())),
                        precision=_HIGH, preferred_element_type=jnp.float32)
    h2s = lax.dot_general(h * h, s_ref[...], (((1,), (0,)), ((), ())),
                          precision=_HIGH, preferred_element_type=jnp.float32)
    nrm8 = jnp.maximum(jnp.sqrt(h2s), 1e-12)
    nrmx = lax.dot_general(nrm8, st_ref[...], (((1,), (0,)), ((), ())),
                           precision=_HIGH, preferred_element_type=jnp.float32)
    hn = h / nrmx
    hn_ref[...] = hn
    tbl_ref[:, :HID] = hn
    tbl_ref[:, HID:] = h


def _prep(x, w_lin):
    R = 1000
    grid = N_NODES // R
    heads_of_ch = jnp.arange(IN_CH, dtype=jnp.int32) // OUT_CH
    sel = (heads_of_ch[:, None] == jnp.arange(HEADS)[None, :])
    S = sel.astype(jnp.float32)
    ST = S.T
    return pl.pallas_call(
        _prep_body,
        grid=(grid,),
        in_specs=[
            pl.BlockSpec((R, IN_CH), lambda i: (i, 0)),
            pl.BlockSpec((HID, IN_CH), lambda i: (0, 0)),
            pl.BlockSpec((IN_CH, HEADS), lambda i: (0, 0)),
            pl.BlockSpec((HEADS, IN_CH), lambda i: (0, 0)),
        ],
        out_specs=[
            pl.BlockSpec((R, HID), lambda i: (i, 0)),
            pl.BlockSpec((R, TD), lambda i: (i, 0)),
        ],
        out_shape=[
            jax.ShapeDtypeStruct((N_NODES, HID), jnp.float32),
            jax.ShapeDtypeStruct((N_NODES, TD), jnp.float32),
        ],
    )(x, w_lin, S, ST)


def _edge_body(tbl_hbm, hn_hbm, row_hbm, col_hbm, out_hbm,
               idxr, idxc, scidx, vs, vd, msg, acc, semi, semg, semd, semc):
    cid = lax.axis_index("c")
    sid = lax.axis_index("s")
    wid = cid * NS + sid

    z16 = jnp.zeros((16,), jnp.float32)

    def zb(r, _):
        for hh in range(HID // 16):
            msg[0, r, pl.ds(hh * 16, 16)] = z16
        return 0

    lax.fori_loop(0, CH, zb, 0)
    for j in range(ROWS_PT // CH):
        pltpu.sync_copy(msg.at[0], acc.at[pl.ds(sid * ROWS_PT + j * CH, CH)])
    plsc.subcore_barrier()

    lanes = lax.iota(jnp.int32, 16)
    bfly = [lanes ^ (1 << k) for k in range(4)]

    def issue_idx(i, b):
        pltpu.async_copy(row_hbm.at[wid, i], idxr.at[b], semi.at[b])
        pltpu.async_copy(col_hbm.at[wid, i], idxc.at[b], semi.at[b])

    def wait_idx(i, b):
        pltpu.make_async_copy(row_hbm.at[wid, i], idxr.at[b],
                              semi.at[b]).wait()
        pltpu.make_async_copy(col_hbm.at[wid, i], idxc.at[b],
                              semi.at[b]).wait()

    def issue_gathers(b):
        pltpu.async_copy(tbl_hbm.at[idxc.at[b, 0]], vs.at[b], semg.at[b])
        pltpu.async_copy(hn_hbm.at[idxr.at[b, 0]], vd.at[b], semd.at[b])

    def wait_gathers(b):
        pltpu.make_async_copy(tbl_hbm.at[idxc.at[b, 0]], vs.at[b],
                              semg.at[b]).wait()
        pltpu.make_async_copy(hn_hbm.at[idxr.at[b, 0]], vd.at[b],
                              semd.at[b]).wait()

    def compute(b):
        @plsc.parallel_loop(0, CH, 1, unroll=4)
        def ebody(e):
            for hh in range(HEADS):
                sv = vs[b, e, pl.ds(hh * 16, 16)]
                dv = vd[b, e, pl.ds(hh * 16, 16)]
                hv = vs[b, e, pl.ds(HID + hh * 16, 16)]
                cos = sv * dv
                for idx in bfly:
                    cos = cos + _shuffle(cos, idx)
                scale = jnp.minimum(jnp.maximum(cos, EPS), 1.0)
                msg[b, e, pl.ds(hh * 16, 16)] = hv * scale

    issue_idx(0, 0)
    wait_idx(0, 0)
    issue_gathers(0)

    def chunk_pair(j, _):
        for b in (0, 1):
            i = 2 * j + b
            b1 = 1 - b
            if b == 0:
                issue_idx(i + 1, b1)
                wait_gathers(b)
                wait_idx(i + 1, b1)
                issue_gathers(b1)
            else:
                @pl.when(j < NCHUNK // 2 - 1)
                def _():
                    issue_idx(i + 1, b1)
                    wait_idx(i + 1, b1)
                    issue_gathers(b1)
                wait_gathers(b)

            @pl.when(j > 0)
            def _():
                pltpu.make_async_copy(msg.at[b], acc.at[scidx.at[b, 0]],
                                      semc.at[b]).wait()
            for o in (0, 16, 24):
                scidx[b, 0, pl.ds(o, 16)] = idxr[b, 0, pl.ds(o, 16)]

            compute(b)
            pltpu.async_copy(msg.at[b], acc.at[scidx.at[b, 0]], semc.at[b],
                             add=True)
        return 0

    lax.fori_loop(0, NCHUNK // 2, chunk_pair, 0)
    for b in (0, 1):
        pltpu.make_async_copy(msg.at[b], acc.at[scidx.at[b, 0]],
                              semc.at[b]).wait()
    plsc.subcore_barrier()
    pltpu.sync_copy(acc.at[pl.ds(sid * ROWS_PT, ROWS_PT)],
                    out_hbm.at[cid, pl.ds(sid * ROWS_PT, ROWS_PT)])


def _edge(tbl, hn, row_idx, col_idx):
    mesh = plsc.VectorSubcoreMesh(core_axis_name="c", subcore_axis_name="s")
    f = functools.partial(
        pl.kernel,
        out_type=jax.ShapeDtypeStruct((NC, N_PAD, HID), jnp.float32),
        mesh=mesh,
        scratch_types=[
            pltpu.VMEM((2, 1, CH), jnp.int32),
            pltpu.VMEM((2, 1, CH), jnp.int32),
            pltpu.VMEM((2, 1, CH), jnp.int32),
            pltpu.VMEM((2, CH, TD), jnp.float32),
            pltpu.VMEM((2, CH, HID), jnp.float32),
            pltpu.VMEM((2, CH, HID), jnp.float32),
            pltpu.VMEM_SHARED((N_PAD, HID), jnp.float32),
            pltpu.SemaphoreType.DMA((2,)),
            pltpu.SemaphoreType.DMA((2,)),
            pltpu.SemaphoreType.DMA((2,)),
            pltpu.SemaphoreType.DMA((2,)),
        ],
    )(_edge_body)
    return f(tbl, hn, row_idx, col_idx)


def _fin_body(p_ref, w_ref, g_ref, b_ref, o_ref):
    oc = p_ref[0] + p_ref[1]
    w = w_ref[...]
    lin = lax.dot_general(oc, w, (((1,), (1,)), ((), ())),
                          precision=_HIGH, preferred_element_type=jnp.float32)
    ones = jnp.ones((1, HID), jnp.float32)
    wn2 = lax.dot_general(ones, w * w, (((1,), (1,)), ((), ())),
                          precision=_HIGH, preferred_element_type=jnp.float32)
    winv = 1.0 / jnp.maximum(jnp.sqrt(wn2), 1e-12)
    xn2 = jnp.sum(oc * oc, axis=1, keepdims=True)
    xinv = 1.0 / jnp.maximum(jnp.sqrt(xn2), 1e-12)
    cos2 = jnp.clip(lin * xinv * winv, EPS, 1.0)
    ob = lin * cos2
    mu = jnp.mean(ob, axis=1, keepdims=True)
    d = ob - mu
    var = jnp.mean(d * d, axis=1, keepdims=True)
    o_ref[...] = d * lax.rsqrt(var + LN_EPS) * g_ref[...] + b_ref[...]


def _finish(partials, w_bcos, g, b):
    R = 1000
    grid = N_NODES // R
    return pl.pallas_call(
        _fin_body,
        grid=(grid,),
        in_specs=[
            pl.BlockSpec((NC, R, HID), lambda i: (0, i, 0)),
            pl.BlockSpec((HID, HID), lambda i: (0, 0)),
            pl.BlockSpec((1, HID), lambda i: (0, 0)),
            pl.BlockSpec((1, HID), lambda i: (0, 0)),
        ],
        out_specs=pl.BlockSpec((R, HID), lambda i: (i, 0)),
        out_shape=jax.ShapeDtypeStruct((N_NODES, HID), jnp.float32),
    )(partials, w_bcos, g, b)


def kernel(x, edge_index, W_lin, W_bcos, ln_gamma, ln_beta):
    hn, tbl = _prep(x, W_lin)
    row4 = edge_index[0].reshape(NW, NCHUNK, 1, CH)
    col4 = edge_index[1].reshape(NW, NCHUNK, 1, CH)
    partials = _edge(tbl, hn, row4, col4)
    return _finish(partials, W_bcos,
                   ln_gamma.reshape(1, HID), ln_beta.reshape(1, HID))


# speedup vs baseline: 2.5630x; 1.0820x over previous
import functools

import jax
import jax.numpy as jnp
from jax import lax
from jax.experimental import pallas as pl
from jax.experimental.pallas import tpu as pltpu
from jax.experimental.pallas import tpu_sc as plsc

N_NODES = 10000
N_EDGES = 320000
IN_CH = 128
HEADS = 8
OUT_CH = 16
HID = HEADS * OUT_CH
EPS = 1e-06
LN_EPS = 1e-05

NC = 2
NS = 16
NW = NC * NS
EPT = N_EDGES // NW
CH = 40
NCHUNK = EPT // CH
N_PAD = 10240
ROWS_PT = N_PAD // NS
ZROWS = 64
TD = 2 * HID

_HIGH = jax.lax.Precision.HIGHEST

_GDN = lax.GatherDimensionNumbers(
    offset_dims=(), collapsed_slice_dims=(0,), start_index_map=(0,))


def _shuffle(v, idx):
    return lax.gather(v, idx[:, None], _GDN, (1,),
                      mode=lax.GatherScatterMode.PROMISE_IN_BOUNDS)


def _prep_body(x_ref, wlin_ref, s_ref, st_ref, hn_ref, tbl_ref):
    x = x_ref[...]
    h = lax.dot_general(x, wlin_ref[...], (((1,), (1,)), ((), ())),
                        precision=_HIGH, preferred_element_type=jnp.float32)
    h2s = lax.dot_general(h * h, s_ref[...], (((1,), (0,)), ((), ())),
                          precision=_HIGH, preferred_element_type=jnp.float32)
    nrm8 = jnp.maximum(jnp.sqrt(h2s), 1e-12)
    nrmx = lax.dot_general(nrm8, st_ref[...], (((1,), (0,)), ((), ())),
                           precision=_HIGH, preferred_element_type=jnp.float32)
    hn = h / nrmx
    hn_ref[...] = hn
    tbl_ref[:, :HID] = hn
    tbl_ref[:, HID:] = h


def _prep(x, w_lin):
    R = 1000
    grid = N_NODES // R
    heads_of_ch = jnp.arange(IN_CH, dtype=jnp.int32) // OUT_CH
    sel = (heads_of_ch[:, None] == jnp.arange(HEADS)[None, :])
    S = sel.astype(jnp.float32)
    ST = S.T
    return pl.pallas_call(
        _prep_body,
        grid=(grid,),
        in_specs=[
            pl.BlockSpec((R, IN_CH), lambda i: (i, 0)),
            pl.BlockSpec((HID, IN_CH), lambda i: (0, 0)),
            pl.BlockSpec((IN_CH, HEADS), lambda i: (0, 0)),
            pl.BlockSpec((HEADS, IN_CH), lambda i: (0, 0)),
        ],
        out_specs=[
            pl.BlockSpec((R, HID), lambda i: (i, 0)),
            pl.BlockSpec((R, TD), lambda i: (i, 0)),
        ],
        out_shape=[
            jax.ShapeDtypeStruct((N_NODES, HID), jnp.float32),
            jax.ShapeDtypeStruct((N_NODES, TD), jnp.float32),
        ],
    )(x, w_lin, S, ST)


def _edge_body(tbl_hbm, hn_hbm, row_hbm, col_hbm, out_hbm,
               idxr, idxc, scidx, vs, vd, msg, acc, semi, semg, semd, semc):
    cid = lax.axis_index("c")
    sid = lax.axis_index("s")
    wid = cid * NS + sid

    z16 = jnp.zeros((16,), jnp.float32)

    def zb(r, _):
        for hh in range(HID // 16):
            msg[0, r, pl.ds(hh * 16, 16)] = z16
        return 0

    lax.fori_loop(0, CH, zb, 0)
    for j in range(ROWS_PT // CH):
        pltpu.sync_copy(msg.at[0], acc.at[pl.ds(sid * ROWS_PT + j * CH, CH)])
    plsc.subcore_barrier()

    lanes = lax.iota(jnp.int32, 16)
    x8, x4, x2, x1 = (lanes ^ 8, lanes ^ 4, lanes ^ 2, lanes ^ 1)
    m8 = lanes < 8
    m4 = (lanes & 4) == 0
    m2 = (lanes & 2) == 0
    lane_of = (0, 8, 4, 12, 2, 10, 6, 14)

    def issue_idx(i, b):
        pltpu.async_copy(row_hbm.at[wid, i], idxr.at[b], semi.at[b])
        pltpu.async_copy(col_hbm.at[wid, i], idxc.at[b], semi.at[b])

    def wait_idx(i, b):
        pltpu.make_async_copy(row_hbm.at[wid, i], idxr.at[b],
                              semi.at[b]).wait()
        pltpu.make_async_copy(col_hbm.at[wid, i], idxc.at[b],
                              semi.at[b]).wait()

    def issue_gathers(b):
        pltpu.async_copy(tbl_hbm.at[idxc.at[b, 0]], vs.at[b], semg.at[b])
        pltpu.async_copy(hn_hbm.at[idxr.at[b, 0]], vd.at[b], semd.at[b])

    def wait_gathers(b):
        pltpu.make_async_copy(tbl_hbm.at[idxc.at[b, 0]], vs.at[b],
                              semg.at[b]).wait()
        pltpu.make_async_copy(hn_hbm.at[idxr.at[b, 0]], vd.at[b],
                              semd.at[b]).wait()

    def compute(b):
        @plsc.parallel_loop(0, CH, 1, unroll=2)
        def ebody(e):
            p = [vs[b, e, pl.ds(hh * 16, 16)] * vd[b, e, pl.ds(hh * 16, 16)]
                 for hh in range(HEADS)]
            r = [x + _shuffle(x, x8) for x in p]
            q = [jnp.where(m8, r[2 * k], r[2 * k + 1]) for k in range(4)]
            q = [x + _shuffle(x, x4) for x in q]
            u = [jnp.where(m4, q[0], q[1]), jnp.where(m4, q[2], q[3])]
            u = [x + _shuffle(x, x2) for x in u]
            v = jnp.where(m2, u[0], u[1])
            v = v + _shuffle(v, x1)
            v = jnp.minimum(jnp.maximum(v, EPS), 1.0)
            for hh in range(HEADS):
                hv = vs[b, e, pl.ds(HID + hh * 16, 16)]
                msg[b, e, pl.ds(hh * 16, 16)] = hv * v[lane_of[hh]]

    issue_idx(0, 0)
    wait_idx(0, 0)
    issue_gathers(0)

    def chunk_pair(j, _):
        for b in (0, 1):
            i = 2 * j + b
            b1 = 1 - b
            if b == 0:
                issue_idx(i + 1, b1)
                wait_gathers(b)
                wait_idx(i + 1, b1)
                issue_gathers(b1)
            else:
                @pl.when(j < NCHUNK // 2 - 1)
                def _():
                    issue_idx(i + 1, b1)
                    wait_idx(i + 1, b1)
                    issue_gathers(b1)
                wait_gathers(b)

            @pl.when(j > 0)
            def _():
                pltpu.make_async_copy(msg.at[b], acc.at[scidx.at[b, 0]],
                                      semc.at[b]).wait()
            for o in (0, 16, 24):
                scidx[b, 0, pl.ds(o, 16)] = idxr[b, 0, pl.ds(o, 16)]

            compute(b)
            pltpu.async_copy(msg.at[b], acc.at[scidx.at[b, 0]], semc.at[b],
                             add=True)
        return 0

    lax.fori_loop(0, NCHUNK // 2, chunk_pair, 0)
    for b in (0, 1):
        pltpu.make_async_copy(msg.at[b], acc.at[scidx.at[b, 0]],
                              semc.at[b]).wait()
    plsc.subcore_barrier()
    pltpu.sync_copy(acc.at[pl.ds(sid * ROWS_PT, ROWS_PT)],
                    out_hbm.at[cid, pl.ds(sid * ROWS_PT, ROWS_PT)])


def _edge(tbl, hn, row_idx, col_idx):
    mesh = plsc.VectorSubcoreMesh(core_axis_name="c", subcore_axis_name="s")
    f = functools.partial(
        pl.kernel,
        out_type=jax.ShapeDtypeStruct((NC, N_PAD, HID), jnp.float32),
        mesh=mesh,
        scratch_types=[
            pltpu.VMEM((2, 1, CH), jnp.int32),
            pltpu.VMEM((2, 1, CH), jnp.int32),
            pltpu.VMEM((2, 1, CH), jnp.int32),
            pltpu.VMEM((2, CH, TD), jnp.float32),
            pltpu.VMEM((2, CH, HID), jnp.float32),
            pltpu.VMEM((2, CH, HID), jnp.float32),
            pltpu.VMEM_SHARED((N_PAD, HID), jnp.float32),
            pltpu.SemaphoreType.DMA((2,)),
            pltpu.SemaphoreType.DMA((2,)),
            pltpu.SemaphoreType.DMA((2,)),
            pltpu.SemaphoreType.DMA((2,)),
        ],
    )(_edge_body)
    return f(tbl, hn, row_idx, col_idx)


def _fin_body(p_ref, w_ref, g_ref, b_ref, o_ref):
    oc = p_ref[0] + p_ref[1]
    w = w_ref[...]
    lin = lax.dot_general(oc, w, (((1,), (1,)), ((), ())),
                          precision=_HIGH, preferred_element_type=jnp.float32)
    ones = jnp.ones((1, HID), jnp.float32)
    wn2 = lax.dot_general(ones, w * w, (((1,), (1,)), ((), ())),
                          precision=_HIGH, preferred_element_type=jnp.float32)
    winv = 1.0 / jnp.maximum(jnp.sqrt(wn2), 1e-12)
    xn2 = jnp.sum(oc * oc, axis=1, keepdims=True)
    xinv = 1.0 / jnp.maximum(jnp.sqrt(xn2), 1e-12)
    cos2 = jnp.clip(lin * xinv * winv, EPS, 1.0)
    ob = lin * cos2
    mu = jnp.mean(ob, axis=1, keepdims=True)
    d = ob - mu
    var = jnp.mean(d * d, axis=1, keepdims=True)
    o_ref[...] = d * lax.rsqrt(var + LN_EPS) * g_ref[...] + b_ref[...]


def _finish(partials, w_bcos, g, b):
    R = 1000
    grid = N_NODES // R
    return pl.pallas_call(
        _fin_body,
        grid=(grid,),
        in_specs=[
            pl.BlockSpec((NC, R, HID), lambda i: (0, i, 0)),
            pl.BlockSpec((HID, HID), lambda i: (0, 0)),
            pl.BlockSpec((1, HID), lambda i: (0, 0)),
            pl.BlockSpec((1, HID), lambda i: (0, 0)),
        ],
        out_specs=pl.BlockSpec((R, HID), lambda i: (i, 0)),
        out_shape=jax.ShapeDtypeStruct((N_NODES, HID), jnp.float32),
    )(partials, w_bcos, g, b)


def kernel(x, edge_index, W_lin, W_bcos, ln_gamma, ln_beta):
    hn, tbl = _prep(x, W_lin)
    row4 = edge_index[0].reshape(NW, NCHUNK, 1, CH)
    col4 = edge_index[1].reshape(NW, NCHUNK, 1, CH)
    partials = _edge(tbl, hn, row4, col4)
    return _finish(partials, W_bcos,
                   ln_gamma.reshape(1, HID), ln_beta.reshape(1, HID))
